# baseline (speedup 1.0000x reference)
import functools

import jax
import jax.numpy as jnp
from jax import lax
from jax.experimental import pallas as pl
from jax.experimental.pallas import tpu as pltpu
from jax.experimental.pallas import tpu_sc as plsc

NUM_USERS = 100000
NUM_ITEMS = 100000
D = 64
B = 4096

NC = 2
NS = 16
L = 16
NW = NC * NS
BPW = B // NW
CH = D // L
GROUPS = BPW // L


def _gmf_body(uid_hbm, iid_hbm, ut_hbm, it_hbm, w_hbm, bb_hbm, out_hbm,
              uid_v, iid_v, u_v, v_v, w_v, bb_v, o_v, sem_u, sem_v):
    wid = lax.axis_index("s") * NC + lax.axis_index("c")
    base = wid * BPW

    pltpu.sync_copy(uid_hbm.at[pl.ds(base, BPW)], uid_v)
    pltpu.sync_copy(iid_hbm.at[pl.ds(base, BPW)], iid_v)
    cp_u = pltpu.async_copy(ut_hbm.at[uid_v], u_v, sem_u)
    cp_v = pltpu.async_copy(it_hbm.at[iid_v], v_v, sem_v)

    pltpu.sync_copy(w_hbm, w_v)
    pltpu.sync_copy(bb_hbm, bb_v)

    cp_u.wait()
    cp_v.wait()

    wc = [w_v[pl.ds(L * c, L)] for c in range(CH)]
    bias = bb_v[...]
    lane = lax.iota(jnp.int32, L)
    masks = [lane == r for r in range(L)]

    for g in range(GROUPS):
        acc = jnp.zeros((L,), jnp.float32)
        for r in range(L):
            row = g * L + r
            part = u_v[row, pl.ds(0, L)] * v_v[row, pl.ds(0, L)] * wc[0]
            for c in range(1, CH):
                part = part + (u_v[row, pl.ds(L * c, L)]
                               * v_v[row, pl.ds(L * c, L)] * wc[c])
            acc = jnp.where(masks[r], jnp.sum(part), acc)
        x = acc + bias
        o_v[pl.ds(g * L, L)] = 1.0 / (1.0 + jnp.exp(-x))

    pltpu.sync_copy(o_v, out_hbm.at[pl.ds(base, BPW)])


@functools.partial(jax.jit, static_argnames=())
def _gmf_sc(user_ids, item_ids, user_table, item_table, w_flat, b_vec):
    mesh = plsc.VectorSubcoreMesh(
        core_axis_name="c", subcore_axis_name="s",
        num_cores=NC, num_subcores=NS)
    run = pl.kernel(
        _gmf_body,
        mesh=mesh,
        compiler_params=pltpu.CompilerParams(
            needs_layout_passes=False, use_tc_tiling_on_sc=False),
        out_type=jax.ShapeDtypeStruct((B,), jnp.float32),
        scratch_types=[
            pltpu.VMEM((BPW,), jnp.int32),
            pltpu.VMEM((BPW,), jnp.int32),
            pltpu.VMEM((BPW, D), jnp.float32),
            pltpu.VMEM((BPW, D), jnp.float32),
            pltpu.VMEM((D,), jnp.float32),
            pltpu.VMEM((L,), jnp.float32),
            pltpu.VMEM((BPW,), jnp.float32),
            pltpu.SemaphoreType.DMA,
            pltpu.SemaphoreType.DMA,
        ],
    )
    return run(user_ids, item_ids, user_table, item_table, w_flat, b_vec)


def kernel(user_ids, item_ids, user_table, item_table, W, b):
    w_flat = W.reshape(D)
    b_vec = jnp.broadcast_to(b.astype(jnp.float32), (L,))
    out = _gmf_sc(user_ids, item_ids, user_table, item_table, w_flat, b_vec)
    return out.reshape(B, 1)


# speedup vs baseline: 1.5176x; 1.5176x over previous
import functools

import jax
import jax.numpy as jnp
from jax import lax
from jax.experimental import pallas as pl
from jax.experimental.pallas import tpu as pltpu
from jax.experimental.pallas import tpu_sc as plsc

NUM_USERS = 100000
NUM_ITEMS = 100000
D = 64
B = 4096

NC = 2
NS = 16
L = 16
NW = NC * NS
BPW = B // NW
CH = D // L
TR = 8
HALF = BPW // 4


def _gmf_body(uid_hbm, iid_hbm, ut_hbm, it_hbm, w_hbm, bb_hbm, out_hbm,
              uid_v, iid_v, ublk_v, iblk_v, u_blk, v_blk, w_v, bb_v, o_v,
              sem_u, sem_v):
    wid = lax.axis_index("s") * NC + lax.axis_index("c")
    base = wid * BPW

    pltpu.sync_copy(uid_hbm.at[pl.ds(base, BPW)], uid_v)
    pltpu.sync_copy(iid_hbm.at[pl.ds(base, BPW)], iid_v)
    for k in range(BPW // L):
        sl = pl.ds(k * L, L)
        ublk_v[sl] = jnp.right_shift(uid_v[sl], 3)
        iblk_v[sl] = jnp.right_shift(iid_v[sl], 3)

    pltpu.sync_copy(w_hbm, w_v)
    pltpu.sync_copy(bb_hbm, bb_v)

    wc = [w_v[pl.ds(L * c, L)] for c in range(CH)]
    bias = bb_v[...]
    lane = lax.iota(jnp.int32, L)
    masks = [lane == r for r in range(L)]

    for h in range(BPW // HALF):
        cps = []
        for g in range(HALF // L):
            ubch = ublk_v[pl.ds(h * HALF + g * L, L)]
            ibch = iblk_v[pl.ds(h * HALF + g * L, L)]
            for r in range(L):
                j = g * L + r
                cps.append(pltpu.async_copy(
                    ut_hbm.at[ubch[r]], u_blk.at[j], sem_u))
                cps.append(pltpu.async_copy(
                    it_hbm.at[ibch[r]], v_blk.at[j], sem_v))
        for cp in cps:
            cp.wait()
        for g in range(HALF // L):
            acc = jnp.zeros((L,), jnp.float32)
            usub = uid_v[pl.ds(h * HALF + g * L, L)] & 7
            isub = iid_v[pl.ds(h * HALF + g * L, L)] & 7
            for r in range(L):
                j = g * L + r
                ru = usub[r]
                rv = isub[r]
                part = (u_blk[j, ru, pl.ds(0, L)]
                        * v_blk[j, rv, pl.ds(0, L)] * wc[0])
                for c in range(1, CH):
                    part = part + (u_blk[j, ru, pl.ds(L * c, L)]
                                   * v_blk[j, rv, pl.ds(L * c, L)] * wc[c])
                acc = jnp.where(masks[r], jnp.sum(part), acc)
            x = acc + bias
            o_v[pl.ds(h * HALF + g * L, L)] = 1.0 / (1.0 + jnp.exp(-x))

    pltpu.sync_copy(o_v, out_hbm.at[pl.ds(base, BPW)])


@jax.jit
def _gmf_sc(user_ids, item_ids, user_table3, item_table3, w_flat, b_vec):
    mesh = plsc.VectorSubcoreMesh(
        core_axis_name="c", subcore_axis_name="s",
        num_cores=NC, num_subcores=NS)
    run = pl.kernel(
        _gmf_body,
        mesh=mesh,
        compiler_params=pltpu.CompilerParams(needs_layout_passes=False),
        out_type=jax.ShapeDtypeStruct((B,), jnp.float32),
        scratch_types=[
            pltpu.VMEM((BPW,), jnp.int32),
            pltpu.VMEM((BPW,), jnp.int32),
            pltpu.VMEM((BPW,), jnp.int32),
            pltpu.VMEM((BPW,), jnp.int32),
            pltpu.VMEM((HALF, TR, D), jnp.float32),
            pltpu.VMEM((HALF, TR, D), jnp.float32),
            pltpu.VMEM((D,), jnp.float32),
            pltpu.VMEM((L,), jnp.float32),
            pltpu.VMEM((BPW,), jnp.float32),
            pltpu.SemaphoreType.DMA,
            pltpu.SemaphoreType.DMA,
        ],
    )
    return run(user_ids, item_ids, user_table3, item_table3, w_flat, b_vec)


def kernel(user_ids, item_ids, user_table, item_table, W, b):
    ut3 = user_table.reshape(NUM_USERS // TR, TR, D)
    it3 = item_table.reshape(NUM_ITEMS // TR, TR, D)
    w_flat = W.reshape(D)
    b_vec = jnp.broadcast_to(b.astype(jnp.float32), (L,))
    out = _gmf_sc(user_ids, item_ids, ut3, it3, w_flat, b_vec)
    return out.reshape(B, 1)
